# baseline (speedup 1.0000x reference)
import functools

import jax
import jax.numpy as jnp
from jax import lax
from jax.experimental import pallas as pl
from jax.experimental.pallas import tpu as pltpu
from jax.experimental.pallas import tpu_sc as plsc

B, S, D = 4, 8192, 1024
ST, SR = S // 8, 8
DTILE, DLANE = D // 128, 128
DT = 8
SCHUNK = 1024
CST = SCHUNK // 8
NCHUNK = S // SCHUNK
NDJ = D // DT
NTASK = B * NDJ
NWORKERS = 32
TPW = NTASK // NWORKERS


def _body(x_hbm, idx_hbm, out_hbm, xbuf, ib0, ib1, ob0, ob1,
          sem_x, si0, si1, so0, so1):
    nc = 2
    wid = lax.axis_index("s") * nc + lax.axis_index("c")

    lane = jnp.arange(16, dtype=jnp.int32)
    col = lane & 7
    half = lane >> 3

    ibufs = (ib0, ib1)
    obufs = (ob0, ob1)
    isems = (si0, si1)
    osems = (so0, so1)

    @pl.loop(0, TPW)
    def _task(t):
        g = wid * TPW + t
        b = g // NDJ
        dj = g % NDJ
        dt = dj // 16
        dc0 = (dj % 16) * 8

        def idx_slice(c):
            return idx_hbm.at[b, pl.ds(c * CST, CST), dt, :, pl.ds(dc0, DT)]

        def out_slice(c):
            return out_hbm.at[b, pl.ds(c * CST, CST), dt, :, pl.ds(dc0, DT)]

        cp_x = pltpu.async_copy(
            x_hbm.at[b, :, dt, :, pl.ds(dc0, DT)], xbuf, sem_x)
        pltpu.async_copy(idx_slice(0), ibufs[0], isems[0])
        cp_x.wait()

        for c in range(NCHUNK):
            p = c % 2
            ib, ob = ibufs[p], obufs[p]
            pltpu.make_async_copy(idx_slice(c), ib, isems[p]).wait()
            if c + 1 < NCHUNK:
                pltpu.async_copy(idx_slice(c + 1), ibufs[(c + 1) % 2],
                                 isems[(c + 1) % 2])
            if c >= 2:
                pltpu.make_async_copy(ob, out_slice(c - 2), osems[p]).wait()

            @plsc.parallel_loop(0, SCHUNK // 2, unroll=4)
            def _rows(i):
                r = half + 2 * i
                rst, rsr = r >> 3, r & 7
                idxv = plsc.load_gather(ib, [rst, rsr, col])
                plsc.store_scatter(
                    ob, [rst, rsr, col],
                    plsc.load_gather(xbuf, [idxv >> 3, idxv & 7, col]))

            pltpu.async_copy(ob, out_slice(c), osems[p])

        pltpu.make_async_copy(obufs[0], out_slice(NCHUNK - 2), osems[0]).wait()
        pltpu.make_async_copy(obufs[1], out_slice(NCHUNK - 1), osems[1]).wait()


@jax.jit
def _index_pool(x5, idx5):
    mesh = plsc.VectorSubcoreMesh(core_axis_name="c", subcore_axis_name="s")
    return pl.kernel(
        _body,
        out_type=jax.ShapeDtypeStruct((B, ST, DTILE, SR, DLANE), jnp.float32),
        mesh=mesh,
        compiler_params=pltpu.CompilerParams(
            use_tc_tiling_on_sc=False, needs_layout_passes=False
        ),
        scratch_types=[
            pltpu.VMEM((ST, SR, DT), jnp.float32),
            pltpu.VMEM((CST, SR, DT), jnp.int32),
            pltpu.VMEM((CST, SR, DT), jnp.int32),
            pltpu.VMEM((CST, SR, DT), jnp.float32),
            pltpu.VMEM((CST, SR, DT), jnp.float32),
            pltpu.SemaphoreType.DMA,
            pltpu.SemaphoreType.DMA,
            pltpu.SemaphoreType.DMA,
            pltpu.SemaphoreType.DMA,
            pltpu.SemaphoreType.DMA,
        ],
    )(x5, idx5)


def kernel(x, index):
    if index.dtype != jnp.int32:
        index = index.astype(jnp.int32)
    x5 = x.reshape(B, ST, SR, DTILE, DLANE).transpose(0, 1, 3, 2, 4)
    idx5 = index.reshape(B, ST, SR, DTILE, DLANE).transpose(0, 1, 3, 2, 4)
    out5 = _index_pool(x5, idx5)
    return out5.transpose(0, 1, 3, 2, 4).reshape(B, S, D)


# speedup vs baseline: 1.0018x; 1.0018x over previous
import functools

import jax
import jax.numpy as jnp
from jax import lax
from jax.experimental import pallas as pl
from jax.experimental.pallas import tpu as pltpu
from jax.experimental.pallas import tpu_sc as plsc

B, S, D = 4, 8192, 1024
ST, SR = S // 8, 8
DTILE, DLANE = D // 128, 128
DT = 8
SCHUNK = 1024
CST = SCHUNK // 8
NCHUNK = S // SCHUNK
NDJ = D // DT
NTASK = B * NDJ
NWORKERS = 32
TPW = NTASK // NWORKERS


def _body(x_hbm, idx_hbm, out_hbm, xbuf, ib0, ib1, ob0, ob1,
          sem_x, si0, si1, so0, so1):
    nc = 2
    wid = lax.axis_index("s") * nc + lax.axis_index("c")

    lane = jnp.arange(16, dtype=jnp.int32)
    col = lane & 7
    half = lane >> 3

    ibufs = (ib0, ib1)
    obufs = (ob0, ob1)
    isems = (si0, si1)
    osems = (so0, so1)

    @pl.loop(0, TPW)
    def _task(t):
        g = wid * TPW + t
        b = g // NDJ
        dj = g % NDJ
        dt = dj // 16
        dc0 = (dj % 16) * 8

        def idx_copies(c, ib):
            return [(idx_hbm.at[b, pl.ds(c * CST, CST), dt, sr, pl.ds(dc0, DT)],
                     ib.at[sr]) for sr in range(SR)]

        def out_copies(c, ob):
            return [(ob.at[sr],
                     out_hbm.at[b, pl.ds(c * CST, CST), dt, sr, pl.ds(dc0, DT)])
                    for sr in range(SR)]

        for sr in range(SR):
            pltpu.async_copy(x_hbm.at[b, :, dt, sr, pl.ds(dc0, DT)],
                             xbuf.at[sr], sem_x)
        for src, dst in idx_copies(0, ibufs[0]):
            pltpu.async_copy(src, dst, isems[0])
        for sr in range(SR):
            pltpu.make_async_copy(x_hbm.at[b, :, dt, sr, pl.ds(dc0, DT)],
                                  xbuf.at[sr], sem_x).wait()

        for c in range(NCHUNK):
            p = c % 2
            ib, ob = ibufs[p], obufs[p]
            for src, dst in idx_copies(c, ib):
                pltpu.make_async_copy(src, dst, isems[p]).wait()
            if c + 1 < NCHUNK:
                for src, dst in idx_copies(c + 1, ibufs[(c + 1) % 2]):
                    pltpu.async_copy(src, dst, isems[(c + 1) % 2])
            if c >= 2:
                for src, dst in out_copies(c - 2, ob):
                    pltpu.make_async_copy(src, dst, osems[p]).wait()

            @plsc.parallel_loop(0, SCHUNK // 2, unroll=4)
            def _rows(i):
                r = half + 2 * i
                rst, rsr = r >> 3, r & 7
                idxv = plsc.load_gather(ib, [rsr, rst, col])
                plsc.store_scatter(
                    ob, [rsr, rst, col],
                    plsc.load_gather(xbuf, [idxv & 7, idxv >> 3, col]))

            for src, dst in out_copies(c, ob):
                pltpu.async_copy(src, dst, osems[p])

        for src, dst in out_copies(NCHUNK - 2, obufs[0]):
            pltpu.make_async_copy(src, dst, osems[0]).wait()
        for src, dst in out_copies(NCHUNK - 1, obufs[1]):
            pltpu.make_async_copy(src, dst, osems[1]).wait()


@jax.jit
def _index_pool(x5, idx5):
    mesh = plsc.VectorSubcoreMesh(core_axis_name="c", subcore_axis_name="s")
    return pl.kernel(
        _body,
        out_type=jax.ShapeDtypeStruct((B, ST, DTILE, SR, DLANE), jnp.float32),
        mesh=mesh,
        compiler_params=pltpu.CompilerParams(
            use_tc_tiling_on_sc=False, needs_layout_passes=False
        ),
        scratch_types=[
            pltpu.VMEM((SR, ST, DT), jnp.float32),
            pltpu.VMEM((SR, CST, DT), jnp.int32),
            pltpu.VMEM((SR, CST, DT), jnp.int32),
            pltpu.VMEM((SR, CST, DT), jnp.float32),
            pltpu.VMEM((SR, CST, DT), jnp.float32),
            pltpu.SemaphoreType.DMA,
            pltpu.SemaphoreType.DMA,
            pltpu.SemaphoreType.DMA,
            pltpu.SemaphoreType.DMA,
            pltpu.SemaphoreType.DMA,
        ],
    )(x5, idx5)


def kernel(x, index):
    if index.dtype != jnp.int32:
        index = index.astype(jnp.int32)
    x5 = x.reshape(B, ST, SR, DTILE, DLANE).transpose(0, 1, 3, 2, 4)
    idx5 = index.reshape(B, ST, SR, DTILE, DLANE).transpose(0, 1, 3, 2, 4)
    out5 = _index_pool(x5, idx5)
    return out5.transpose(0, 1, 3, 2, 4).reshape(B, S, D)


# speedup vs baseline: 1.0559x; 1.0540x over previous
import functools

import jax
import jax.numpy as jnp
from jax import lax
from jax.experimental import pallas as pl
from jax.experimental.pallas import tpu as pltpu
from jax.experimental.pallas import tpu_sc as plsc

B, S, D = 4, 8192, 1024
ST, SR = S // 8, 8
DTILE, DLANE = D // 128, 128
DT = 8
SCHUNK = 1024
CST = SCHUNK // 8
NCHUNK = S // SCHUNK
NBLK = B * DTILE
BPC = NBLK // 2
XPH = 4
XSPST = ST // XPH


def _body(x_hbm, idx_hbm, out_hbm, xsp, xbuf, ib0, ib1, ob0, ob1,
          sem_x, si0, si1, so0, so1):
    c_ax = lax.axis_index("c")
    k = lax.axis_index("s")
    dc0 = k * DT

    lane = jnp.arange(16, dtype=jnp.int32)
    col = lane & 7
    half = lane >> 3

    ibufs = (ib0, ib1)
    obufs = (ob0, ob1)
    isems = (si0, si1)
    osems = (so0, so1)

    @pl.loop(0, BPC)
    def _block(blk):
        beta = c_ax * BPC + blk
        b = beta // DTILE
        dt = beta % DTILE

        def idx_slice(c):
            return idx_hbm.at[b, pl.ds(c * CST, CST), dt, :, pl.ds(dc0, DT)]

        def out_slice(c):
            return out_hbm.at[b, pl.ds(c * CST, CST), dt, :, pl.ds(dc0, DT)]

        pltpu.async_copy(idx_slice(0), ibufs[0], isems[0])

        for h in range(XPH):
            pltpu.async_copy(
                x_hbm.at[b, pl.ds(h * XSPST + k * (XSPST // 16), XSPST // 16),
                         dt, :, :],
                xsp.at[pl.ds(k * (XSPST // 16), XSPST // 16)], sem_x).wait()
            plsc.subcore_barrier()
            pltpu.async_copy(xsp.at[:, :, pl.ds(dc0, DT)],
                             xbuf.at[pl.ds(h * XSPST, XSPST)], sem_x).wait()
            plsc.subcore_barrier()

        for c in range(NCHUNK):
            p = c % 2
            ib, ob = ibufs[p], obufs[p]
            pltpu.make_async_copy(idx_slice(c), ib, isems[p]).wait()
            if c + 1 < NCHUNK:
                pltpu.async_copy(idx_slice(c + 1), ibufs[1 - p], isems[1 - p])
            if c >= 2:
                pltpu.make_async_copy(ob, out_slice(c - 2), osems[p]).wait()

            @plsc.parallel_loop(0, SCHUNK // 2, unroll=4)
            def _rows(i):
                r = half + 2 * i
                rst, rsr = r >> 3, r & 7
                idxv = plsc.load_gather(ib, [rst, rsr, col])
                plsc.store_scatter(
                    ob, [rst, rsr, col],
                    plsc.load_gather(xbuf, [idxv >> 3, idxv & 7, col]))

            pltpu.async_copy(ob, out_slice(c), osems[p])

        pltpu.make_async_copy(obufs[0], out_slice(NCHUNK - 2), osems[0]).wait()
        pltpu.make_async_copy(obufs[1], out_slice(NCHUNK - 1), osems[1]).wait()


@jax.jit
def _index_pool(x5, idx5):
    mesh = plsc.VectorSubcoreMesh(core_axis_name="c", subcore_axis_name="s")
    return pl.kernel(
        _body,
        out_type=jax.ShapeDtypeStruct((B, ST, DTILE, SR, DLANE), jnp.float32),
        mesh=mesh,
        compiler_params=pltpu.CompilerParams(
            use_tc_tiling_on_sc=False, needs_layout_passes=False
        ),
        scratch_types=[
            pltpu.VMEM_SHARED((XSPST, SR, DLANE), jnp.float32),
            pltpu.VMEM((ST, SR, DT), jnp.float32),
            pltpu.VMEM((CST, SR, DT), jnp.int32),
            pltpu.VMEM((CST, SR, DT), jnp.int32),
            pltpu.VMEM((CST, SR, DT), jnp.float32),
            pltpu.VMEM((CST, SR, DT), jnp.float32),
        ] + [pltpu.SemaphoreType.DMA] * 5,
    )(x5, idx5)


def kernel(x, index):
    if index.dtype != jnp.int32:
        index = index.astype(jnp.int32)
    x5 = x.reshape(B, ST, SR, DTILE, DLANE).transpose(0, 1, 3, 2, 4)
    idx5 = index.reshape(B, ST, SR, DTILE, DLANE).transpose(0, 1, 3, 2, 4)
    out5 = _index_pool(x5, idx5)
    return out5.transpose(0, 1, 3, 2, 4).reshape(B, S, D)
